# Initial kernel scaffold; baseline (speedup 1.0000x reference)
import functools

import jax
import jax.numpy as jnp
from jax import lax
from jax.experimental import pallas as pl
from jax.experimental.pallas import tpu as pltpu
from jax.experimental.pallas import tpu_sc as plsc

_NC = 2
_NS = 16
_NW = _NC * _NS

_K = 128
_DW = 16
_CH = 128


def _sc_degree(dst2d, zeros_hbm, ones_hbm, n_acc, nb):
    rows_pt = n_acc // _NS
    nchunk = rows_pt // _CH
    mesh = plsc.VectorSubcoreMesh(core_axis_name="c", subcore_axis_name="s")

    @functools.partial(
        pl.kernel,
        out_type=jax.ShapeDtypeStruct((_NC, n_acc, _DW), jnp.float32),
        mesh=mesh,
        scratch_types=[
            pltpu.VMEM((nb, _K), jnp.int32),
            pltpu.VMEM((_K, _DW), jnp.float32),
            pltpu.VMEM((_CH, _DW), jnp.float32),
            pltpu.VMEM_SHARED((n_acc, _DW), jnp.float32),
        ],
    )
    def deg_kernel(dst_hbm, z_hbm, o_hbm, out_hbm, dst_v, ones_v, stage_v, dacc):
        c = lax.axis_index("c")
        s = lax.axis_index("s")
        wid = c * _NS + s
        base = s * rows_pt
        pltpu.sync_copy(z_hbm, stage_v)
        for j in range(nchunk):
            pltpu.sync_copy(stage_v, dacc.at[pl.ds(base + j * _CH, _CH)])
        pltpu.sync_copy(o_hbm, ones_v)
        pltpu.sync_copy(dst_hbm.at[pl.ds(wid * nb, nb)], dst_v)
        plsc.subcore_barrier()

        def body(j, carry):
            pltpu.sync_copy(ones_v, dacc.at[dst_v.at[j]], add=True)
            return carry

        lax.fori_loop(0, nb, body, 0)
        plsc.subcore_barrier()
        for j in range(nchunk):
            pltpu.sync_copy(dacc.at[pl.ds(base + j * _CH, _CH)], stage_v)
            pltpu.sync_copy(stage_v, out_hbm.at[c, pl.ds(base + j * _CH, _CH)])

    return deg_kernel(dst2d, zeros_hbm, ones_hbm)


def _sc_aggregate(h2, src2d, dst2d, zeros_hbm, n_acc, nb, d):
    rows_pt = n_acc // _NS
    nchunk = rows_pt // _CH
    mesh = plsc.VectorSubcoreMesh(core_axis_name="c", subcore_axis_name="s")

    @functools.partial(
        pl.kernel,
        out_type=jax.ShapeDtypeStruct((_NC, n_acc, d), jnp.float32),
        mesh=mesh,
        scratch_types=[
            pltpu.VMEM((nb, _K), jnp.int32),
            pltpu.VMEM((nb, _K), jnp.int32),
            pltpu.VMEM((_K, d), jnp.float32),
            pltpu.VMEM_SHARED((n_acc, d), jnp.float32),
            pltpu.SemaphoreType.DMA,
        ],
    )
    def agg_kernel(h2_hbm, src_hbm, dst_hbm, z_hbm, out_hbm,
                   src_v, dst_v, rows_v, acc, sem):
        c = lax.axis_index("c")
        s = lax.axis_index("s")
        wid = c * _NS + s
        base = s * rows_pt
        pltpu.sync_copy(z_hbm, rows_v)
        for j in range(nchunk):
            pltpu.sync_copy(rows_v, acc.at[pl.ds(base + j * _CH, _CH)])
        pltpu.sync_copy(src_hbm.at[pl.ds(wid * nb, nb)], src_v)
        pltpu.sync_copy(dst_hbm.at[pl.ds(wid * nb, nb)], dst_v)
        plsc.subcore_barrier()

        def body(j, carry):
            pltpu.async_copy(h2_hbm.at[src_v.at[j]], rows_v, sem).wait()
            pltpu.sync_copy(rows_v, acc.at[dst_v.at[j]], add=True)
            return carry

        lax.fori_loop(0, nb, body, 0)
        plsc.subcore_barrier()
        for j in range(nchunk):
            pltpu.sync_copy(acc.at[pl.ds(base + j * _CH, _CH)], rows_v)
            pltpu.sync_copy(rows_v, out_hbm.at[c, pl.ds(base + j * _CH, _CH)])

    return agg_kernel(h2, src2d, dst2d, zeros_hbm)


def _tc_h2(x_pad, W, deg_part, n_acc, d, br):
    grid = n_acc // br

    def body(x_ref, w_ref, dp_ref, h2_ref):
        deg = dp_ref[0, :, 0:1] + dp_ref[1, :, 0:1] + 1.0
        dis = lax.rsqrt(deg)
        h = jnp.dot(x_ref[...], w_ref[...], preferred_element_type=jnp.float32)
        h2_ref[...] = h * dis

    return pl.pallas_call(
        body,
        grid=(grid,),
        in_specs=[
            pl.BlockSpec((br, d), lambda i: (i, 0)),
            pl.BlockSpec((d, d), lambda i: (0, 0)),
            pl.BlockSpec((_NC, br, _DW), lambda i: (0, i, 0)),
        ],
        out_specs=pl.BlockSpec((br, d), lambda i: (i, 0)),
        out_shape=jax.ShapeDtypeStruct((n_acc, d), jnp.float32),
    )(x_pad, W, deg_part)


def _tc_combine(agg_part, h2, deg_part, b2, n_acc, d, br):
    grid = n_acc // br

    def body(a_ref, h2_ref, dp_ref, b_ref, o_ref):
        deg = dp_ref[0, :, 0:1] + dp_ref[1, :, 0:1] + 1.0
        dis = lax.rsqrt(deg)
        t = (a_ref[0] + a_ref[1] + h2_ref[...]) * dis + b_ref[...]
        o_ref[...] = jnp.maximum(t, 0.0)

    return pl.pallas_call(
        body,
        grid=(grid,),
        in_specs=[
            pl.BlockSpec((_NC, br, d), lambda i: (0, i, 0)),
            pl.BlockSpec((br, d), lambda i: (i, 0)),
            pl.BlockSpec((_NC, br, _DW), lambda i: (0, i, 0)),
            pl.BlockSpec((1, d), lambda i: (0, 0)),
        ],
        out_specs=pl.BlockSpec((br, d), lambda i: (i, 0)),
        out_shape=jax.ShapeDtypeStruct((n_acc, d), jnp.float32),
    )(agg_part, h2, deg_part, b2)


@jax.jit
def kernel(x, edge_index, W, b):
    n, d = x.shape
    e = edge_index.shape[1]

    nb = -(-e // (_NW * _K))
    e_pad = _NW * nb * _K
    n_acc = -(-(n + 1) // (_NS * _CH)) * (_NS * _CH)
    br = 512

    src = edge_index[0].astype(jnp.int32)
    dst = edge_index[1].astype(jnp.int32)
    pad = e_pad - e
    src2d = jnp.concatenate([src, jnp.zeros((pad,), jnp.int32)]).reshape(_NW * nb, _K)
    dst2d = jnp.concatenate([dst, jnp.full((pad,), n, jnp.int32)]).reshape(_NW * nb, _K)
    x_pad = jnp.pad(x, ((0, n_acc - n), (0, 0)))
    zeros_d = jnp.zeros((_CH, d), jnp.float32)
    zeros_w = jnp.zeros((_CH, _DW), jnp.float32)
    ones_w = jnp.ones((_K, _DW), jnp.float32)
    b2 = b.reshape(1, d).astype(jnp.float32)

    deg_part = _sc_degree(dst2d, zeros_w, ones_w, n_acc, nb)
    h2 = _tc_h2(x_pad, W, deg_part, n_acc, d, br)
    agg_part = _sc_aggregate(h2, src2d, dst2d, zeros_d, n_acc, nb, d)
    out = _tc_combine(agg_part, h2, deg_part, b2, n_acc, d, br)
    return out[:n]


# speedup vs baseline: 14.0461x; 14.0461x over previous
import functools

import jax
import jax.numpy as jnp
from jax import lax
from jax.experimental import pallas as pl
from jax.experimental.pallas import tpu as pltpu
from jax.experimental.pallas import tpu_sc as plsc

_NC = 2
_NS = 16
_NW = _NC * _NS

_K = 128
_DW = 16
_CH = 128


def _sc_degree(dst2d, zeros_hbm, ones_hbm, n_acc, nb):
    rows_pt = n_acc // _NS
    nchunk = rows_pt // _CH
    mesh = plsc.VectorSubcoreMesh(core_axis_name="c", subcore_axis_name="s")

    @functools.partial(
        pl.kernel,
        out_type=jax.ShapeDtypeStruct((_NC, n_acc, _DW), jnp.float32),
        mesh=mesh,
        scratch_types=[
            pltpu.VMEM((nb, _K), jnp.int32),
            pltpu.VMEM((_K, _DW), jnp.float32),
            pltpu.VMEM((_CH, _DW), jnp.float32),
            pltpu.VMEM_SHARED((n_acc, _DW), jnp.float32),
        ],
    )
    def deg_kernel(dst_hbm, z_hbm, o_hbm, out_hbm, dst_v, ones_v, stage_v, dacc):
        c = lax.axis_index("c")
        s = lax.axis_index("s")
        wid = c * _NS + s
        base = s * rows_pt
        pltpu.sync_copy(z_hbm, stage_v)
        for j in range(nchunk):
            pltpu.sync_copy(stage_v, dacc.at[pl.ds(base + j * _CH, _CH)])
        pltpu.sync_copy(o_hbm, ones_v)
        pltpu.sync_copy(dst_hbm.at[pl.ds(wid * nb, nb)], dst_v)
        plsc.subcore_barrier()

        def body(j, carry):
            pltpu.sync_copy(ones_v, dacc.at[dst_v.at[j]], add=True)
            return carry

        lax.fori_loop(0, nb, body, 0)
        plsc.subcore_barrier()
        for j in range(nchunk):
            pltpu.sync_copy(dacc.at[pl.ds(base + j * _CH, _CH)], stage_v)
            pltpu.sync_copy(stage_v, out_hbm.at[c, pl.ds(base + j * _CH, _CH)])

    return deg_kernel(dst2d, zeros_hbm, ones_hbm)


def _sc_aggregate(h2, src2d, dst2d, zeros_hbm, n_acc, nb, d):
    rows_pt = n_acc // _NS
    nchunk = rows_pt // _CH
    mesh = plsc.VectorSubcoreMesh(core_axis_name="c", subcore_axis_name="s")

    @functools.partial(
        pl.kernel,
        out_type=jax.ShapeDtypeStruct((_NC, n_acc, d), jnp.float32),
        mesh=mesh,
        scratch_types=[
            pltpu.VMEM((nb, _K), jnp.int32),
            pltpu.VMEM((nb, _K), jnp.int32),
            pltpu.VMEM((_K, d), jnp.float32),
            pltpu.VMEM_SHARED((n_acc, d), jnp.float32),
            pltpu.SemaphoreType.DMA,
        ],
    )
    def agg_kernel(h2_hbm, src_hbm, dst_hbm, z_hbm, out_hbm,
                   src_v, dst_v, rows_v, acc, sem):
        c = lax.axis_index("c")
        s = lax.axis_index("s")
        wid = c * _NS + s
        base = s * rows_pt
        pltpu.sync_copy(z_hbm, rows_v)
        for j in range(nchunk):
            pltpu.sync_copy(rows_v, acc.at[pl.ds(base + j * _CH, _CH)])
        pltpu.sync_copy(src_hbm.at[pl.ds(wid * nb, nb)], src_v)
        pltpu.sync_copy(dst_hbm.at[pl.ds(wid * nb, nb)], dst_v)
        plsc.subcore_barrier()

        def body(j, carry):
            pltpu.async_copy(h2_hbm.at[src_v.at[j]], rows_v, sem).wait()
            pltpu.sync_copy(rows_v, acc.at[dst_v.at[j]], add=True)
            return carry

        lax.fori_loop(0, nb, body, 0)
        plsc.subcore_barrier()
        for j in range(nchunk):
            pltpu.sync_copy(acc.at[pl.ds(base + j * _CH, _CH)], rows_v)
            pltpu.sync_copy(rows_v, out_hbm.at[c, pl.ds(base + j * _CH, _CH)])

    return agg_kernel(h2, src2d, dst2d, zeros_hbm)


def _tc_h2(x_pad, W, deg_part, n_acc, d, br):
    grid = n_acc // br

    def body(x_ref, w_ref, dp_ref, h2_ref):
        deg = dp_ref[0, :, 0:1] + dp_ref[1, :, 0:1] + 1.0
        dis = lax.rsqrt(deg)
        h = jnp.dot(x_ref[...], w_ref[...], preferred_element_type=jnp.float32)
        h2_ref[...] = h * dis

    return pl.pallas_call(
        body,
        grid=(grid,),
        in_specs=[
            pl.BlockSpec((br, d), lambda i: (i, 0)),
            pl.BlockSpec((d, d), lambda i: (0, 0)),
            pl.BlockSpec((_NC, br, _DW), lambda i: (0, i, 0)),
        ],
        out_specs=pl.BlockSpec((br, d), lambda i: (i, 0)),
        out_shape=jax.ShapeDtypeStruct((n_acc, d), jnp.float32),
    )(x_pad, W, deg_part)


def _tc_combine(agg_part, h2, deg_part, b2, n_acc, d, br):
    grid = n_acc // br

    def body(a_ref, h2_ref, dp_ref, b_ref, o_ref):
        deg = dp_ref[0, :, 0:1] + dp_ref[1, :, 0:1] + 1.0
        dis = lax.rsqrt(deg)
        t = (a_ref[0] + a_ref[1] + h2_ref[...]) * dis + b_ref[...]
        o_ref[...] = jnp.maximum(t, 0.0)

    return pl.pallas_call(
        body,
        grid=(grid,),
        in_specs=[
            pl.BlockSpec((_NC, br, d), lambda i: (0, i, 0)),
            pl.BlockSpec((br, d), lambda i: (i, 0)),
            pl.BlockSpec((_NC, br, _DW), lambda i: (0, i, 0)),
            pl.BlockSpec((1, d), lambda i: (0, 0)),
        ],
        out_specs=pl.BlockSpec((br, d), lambda i: (i, 0)),
        out_shape=jax.ShapeDtypeStruct((n_acc, d), jnp.float32),
    )(agg_part, h2, deg_part, b2)


@jax.jit
def kernel(x, edge_index, W, b):
    n, d = x.shape
    e = edge_index.shape[1]

    nb = -(-(-(-e // (_NW * _K))) // 8) * 8
    e_pad = _NW * nb * _K
    n_acc = -(-(n + 1) // (_NS * _CH)) * (_NS * _CH)
    br = 512

    src = edge_index[0].astype(jnp.int32)
    dst = edge_index[1].astype(jnp.int32)
    pad = e_pad - e
    src2d = jnp.concatenate([src, jnp.zeros((pad,), jnp.int32)]).reshape(_NW * nb, _K)
    dst2d = jnp.concatenate([dst, jnp.full((pad,), n, jnp.int32)]).reshape(_NW * nb, _K)
    x_pad = jnp.pad(x, ((0, n_acc - n), (0, 0)))
    zeros_d = jnp.zeros((_CH, d), jnp.float32)
    zeros_w = jnp.zeros((_CH, _DW), jnp.float32)
    ones_w = jnp.ones((_K, _DW), jnp.float32)
    b2 = b.reshape(1, d).astype(jnp.float32)

    deg_part = _sc_degree(dst2d, zeros_w, ones_w, n_acc, nb)
    h2 = _tc_h2(x_pad, W, deg_part, n_acc, d, br)
    agg_part = _sc_aggregate(h2, src2d, dst2d, zeros_d, n_acc, nb, d)
    out = _tc_combine(agg_part, h2, deg_part, b2, n_acc, d, br)
    return out[:n]


# speedup vs baseline: 15.9350x; 1.1345x over previous
import functools

import jax
import jax.numpy as jnp
from jax import lax
from jax.experimental import pallas as pl
from jax.experimental.pallas import tpu as pltpu
from jax.experimental.pallas import tpu_sc as plsc

_NC = 2
_NS = 16
_NW = _NC * _NS

_K = 128
_DW = 16
_CH = 128


def _sc_degree(dst2d, zeros_hbm, ones_hbm, n_acc, nb):
    rows_pt = n_acc // _NS
    nchunk = rows_pt // _CH
    mesh = plsc.VectorSubcoreMesh(core_axis_name="c", subcore_axis_name="s")

    @functools.partial(
        pl.kernel,
        out_type=jax.ShapeDtypeStruct((_NC, n_acc, _DW), jnp.float32),
        mesh=mesh,
        scratch_types=[
            pltpu.VMEM((nb, _K), jnp.int32),
            pltpu.VMEM((_K, _DW), jnp.float32),
            pltpu.VMEM((_CH, _DW), jnp.float32),
            pltpu.VMEM_SHARED((n_acc, _DW), jnp.float32),
        ],
    )
    def deg_kernel(dst_hbm, z_hbm, o_hbm, out_hbm, dst_v, ones_v, stage_v, dacc):
        c = lax.axis_index("c")
        s = lax.axis_index("s")
        wid = c * _NS + s
        base = s * rows_pt
        pltpu.sync_copy(z_hbm, stage_v)
        for j in range(nchunk):
            pltpu.sync_copy(stage_v, dacc.at[pl.ds(base + j * _CH, _CH)])
        pltpu.sync_copy(o_hbm, ones_v)
        pltpu.sync_copy(dst_hbm.at[pl.ds(wid * nb, nb)], dst_v)
        plsc.subcore_barrier()

        def body(j, carry):
            pltpu.sync_copy(ones_v, dacc.at[dst_v.at[j]], add=True)
            return carry

        lax.fori_loop(0, nb, body, 0)
        plsc.subcore_barrier()
        for j in range(nchunk):
            pltpu.sync_copy(dacc.at[pl.ds(base + j * _CH, _CH)], stage_v)
            pltpu.sync_copy(stage_v, out_hbm.at[c, pl.ds(base + j * _CH, _CH)])

    return deg_kernel(dst2d, zeros_hbm, ones_hbm)


def _sc_aggregate(h2, src2d, dst2d, zeros_hbm, n_acc, nb, d):
    rows_pt = n_acc // _NS
    nchunk = rows_pt // _CH
    mesh = plsc.VectorSubcoreMesh(core_axis_name="c", subcore_axis_name="s")

    nph = nb // 2

    @functools.partial(
        pl.kernel,
        out_type=jax.ShapeDtypeStruct((_NC, n_acc, d), jnp.float32),
        mesh=mesh,
        scratch_types=[
            pltpu.VMEM((nph, _K), jnp.int32),
            pltpu.VMEM((nph, _K), jnp.int32),
            pltpu.VMEM((_K, d), jnp.float32),
            pltpu.VMEM((_K, d), jnp.float32),
            pltpu.VMEM_SHARED((n_acc, d), jnp.float32),
            pltpu.SemaphoreType.DMA,
            pltpu.SemaphoreType.DMA,
        ],
    )
    def agg_kernel(h2_hbm, src_hbm, dst_hbm, z_hbm, out_hbm,
                   src_v, dst_v, rows_a, rows_b, acc, sem_a, sem_b):
        c = lax.axis_index("c")
        s = lax.axis_index("s")
        wid = c * _NS + s
        base = s * rows_pt
        pltpu.sync_copy(z_hbm, rows_a)
        for j in range(nchunk):
            pltpu.sync_copy(rows_a, acc.at[pl.ds(base + j * _CH, _CH)])
        plsc.subcore_barrier()

        half = nph // 2
        for ph in range(2):
            pltpu.sync_copy(src_hbm.at[pl.ds(wid * nb + ph * nph, nph)], src_v)
            pltpu.sync_copy(dst_hbm.at[pl.ds(wid * nb + ph * nph, nph)], dst_v)
            pltpu.async_copy(h2_hbm.at[src_v.at[0]], rows_a, sem_a)

            def body(j, carry):
                b = 2 * j
                pltpu.async_copy(h2_hbm.at[src_v.at[b + 1]], rows_b, sem_b)
                pltpu.make_async_copy(h2_hbm.at[src_v.at[b]], rows_a, sem_a).wait()
                pltpu.sync_copy(rows_a, acc.at[dst_v.at[b]], add=True)

                @pl.when(j < half - 1)
                def _():
                    pltpu.async_copy(h2_hbm.at[src_v.at[b + 2]], rows_a, sem_a)

                pltpu.make_async_copy(h2_hbm.at[src_v.at[b + 1]], rows_b, sem_b).wait()
                pltpu.sync_copy(rows_b, acc.at[dst_v.at[b + 1]], add=True)
                return carry

            lax.fori_loop(0, half, body, 0)
        plsc.subcore_barrier()
        for j in range(nchunk):
            pltpu.sync_copy(acc.at[pl.ds(base + j * _CH, _CH)], rows_a)
            pltpu.sync_copy(rows_a, out_hbm.at[c, pl.ds(base + j * _CH, _CH)])

    return agg_kernel(h2, src2d, dst2d, zeros_hbm)


def _tc_h2(x_pad, W, deg_part, n_acc, d, br):
    grid = n_acc // br

    def body(x_ref, w_ref, dp_ref, h2_ref):
        deg = dp_ref[0, :, 0:1] + dp_ref[1, :, 0:1] + 1.0
        dis = lax.rsqrt(deg)
        h = jnp.dot(x_ref[...], w_ref[...], preferred_element_type=jnp.float32)
        h2_ref[...] = h * dis

    return pl.pallas_call(
        body,
        grid=(grid,),
        in_specs=[
            pl.BlockSpec((br, d), lambda i: (i, 0)),
            pl.BlockSpec((d, d), lambda i: (0, 0)),
            pl.BlockSpec((_NC, br, _DW), lambda i: (0, i, 0)),
        ],
        out_specs=pl.BlockSpec((br, d), lambda i: (i, 0)),
        out_shape=jax.ShapeDtypeStruct((n_acc, d), jnp.float32),
    )(x_pad, W, deg_part)


def _tc_combine(agg_part, h2, deg_part, b2, n_acc, d, br):
    grid = n_acc // br

    def body(a_ref, h2_ref, dp_ref, b_ref, o_ref):
        deg = dp_ref[0, :, 0:1] + dp_ref[1, :, 0:1] + 1.0
        dis = lax.rsqrt(deg)
        t = (a_ref[0] + a_ref[1] + h2_ref[...]) * dis + b_ref[...]
        o_ref[...] = jnp.maximum(t, 0.0)

    return pl.pallas_call(
        body,
        grid=(grid,),
        in_specs=[
            pl.BlockSpec((_NC, br, d), lambda i: (0, i, 0)),
            pl.BlockSpec((br, d), lambda i: (i, 0)),
            pl.BlockSpec((_NC, br, _DW), lambda i: (0, i, 0)),
            pl.BlockSpec((1, d), lambda i: (0, 0)),
        ],
        out_specs=pl.BlockSpec((br, d), lambda i: (i, 0)),
        out_shape=jax.ShapeDtypeStruct((n_acc, d), jnp.float32),
    )(agg_part, h2, deg_part, b2)


@jax.jit
def kernel(x, edge_index, W, b):
    n, d = x.shape
    e = edge_index.shape[1]

    nb = -(-(-(-e // (_NW * _K))) // 8) * 8
    e_pad = _NW * nb * _K
    n_acc = -(-(n + 1) // (_NS * _CH)) * (_NS * _CH)
    br = 512

    src = edge_index[0].astype(jnp.int32)
    dst = edge_index[1].astype(jnp.int32)
    pad = e_pad - e
    src2d = jnp.concatenate([src, jnp.zeros((pad,), jnp.int32)]).reshape(_NW * nb, _K)
    dst2d = jnp.concatenate([dst, jnp.full((pad,), n, jnp.int32)]).reshape(_NW * nb, _K)
    x_pad = jnp.pad(x, ((0, n_acc - n), (0, 0)))
    zeros_d = jnp.zeros((_CH, d), jnp.float32)
    zeros_w = jnp.zeros((_CH, _DW), jnp.float32)
    ones_w = jnp.ones((_K, _DW), jnp.float32)
    b2 = b.reshape(1, d).astype(jnp.float32)

    deg_part = _sc_degree(dst2d, zeros_w, ones_w, n_acc, nb)
    h2 = _tc_h2(x_pad, W, deg_part, n_acc, d, br)
    agg_part = _sc_aggregate(h2, src2d, dst2d, zeros_d, n_acc, nb, d)
    out = _tc_combine(agg_part, h2, deg_part, b2, n_acc, d, br)
    return out[:n]
